# baseline (speedup 1.0000x reference)
import functools

import jax
import jax.numpy as jnp
import numpy as np
from jax.experimental import pallas as pl
from jax.experimental.pallas import tpu as pltpu

PC_RANGE = [-51.2, -51.2, -5.0, 51.2, 51.2, 3.0]
SPATIAL_SHAPES = [(58, 100), (29, 50), (15, 25), (8, 13)]
EMBED = 256
HEADS = 8
LEVELS = 4
POINTS = 8
NUM_REFS = 4
NUM_CAMS = 6
NQ = 2500
HEAD_DIM = EMBED // HEADS
NUM_VALUE = sum(h * w for h, w in SPATIAL_SHAPES)

NQ_PAD = 2560
QBLK = 128
NV_PAD = 7936
VBLK = 128


def _frontend_body(q_ref, woff_ref, boff_ref, wattn_ref, battn_ref,
                   off_ref, aw_ref):
    qb = q_ref[...]
    off = jnp.dot(qb, woff_ref[...], preferred_element_type=jnp.float32)
    off_ref[...] = off + boff_ref[...]
    lg = jnp.dot(qb, wattn_ref[...], preferred_element_type=jnp.float32)
    lg = lg + battn_ref[...]
    lp = LEVELS * POINTS
    for h in range(HEADS):
        sl = lg[:, h * lp:(h + 1) * lp]
        m = jnp.max(sl, axis=-1, keepdims=True)
        e = jnp.exp(sl - m)
        s = jnp.sum(e, axis=-1, keepdims=True)
        aw_ref[:, h * lp:(h + 1) * lp] = e / s


def _frontend(q_pad, W_off, b_off, W_attn, b_attn):
    grid = (NQ_PAD // QBLK,)
    return pl.pallas_call(
        _frontend_body,
        grid=grid,
        in_specs=[
            pl.BlockSpec((QBLK, EMBED), lambda i: (i, 0)),
            pl.BlockSpec((EMBED, HEADS * LEVELS * POINTS * 2), lambda i: (0, 0)),
            pl.BlockSpec((1, HEADS * LEVELS * POINTS * 2), lambda i: (0, 0)),
            pl.BlockSpec((EMBED, HEADS * LEVELS * POINTS), lambda i: (0, 0)),
            pl.BlockSpec((1, HEADS * LEVELS * POINTS), lambda i: (0, 0)),
        ],
        out_specs=[
            pl.BlockSpec((QBLK, HEADS * LEVELS * POINTS * 2), lambda i: (i, 0)),
            pl.BlockSpec((QBLK, HEADS * LEVELS * POINTS), lambda i: (i, 0)),
        ],
        out_shape=[
            jax.ShapeDtypeStruct((NQ_PAD, HEADS * LEVELS * POINTS * 2), jnp.float32),
            jax.ShapeDtypeStruct((NQ_PAD, HEADS * LEVELS * POINTS), jnp.float32),
        ],
    )(q_pad, W_off, b_off.reshape(1, -1), W_attn, b_attn.reshape(1, -1))


def _valproj_body(v_ref, w_ref, b_ref, o_ref):
    o_ref[0] = jnp.dot(v_ref[0], w_ref[...],
                       preferred_element_type=jnp.float32) + b_ref[...]


def _valproj(value_pad, W_val, b_val):
    grid = (NUM_CAMS, NV_PAD // VBLK)
    return pl.pallas_call(
        _valproj_body,
        grid=grid,
        in_specs=[
            pl.BlockSpec((1, VBLK, EMBED), lambda c, i: (c, i, 0)),
            pl.BlockSpec((EMBED, EMBED), lambda c, i: (0, 0)),
            pl.BlockSpec((1, EMBED), lambda c, i: (0, 0)),
        ],
        out_specs=pl.BlockSpec((1, VBLK, EMBED), lambda c, i: (c, i, 0)),
        out_shape=jax.ShapeDtypeStruct((NUM_CAMS, NV_PAD, EMBED), jnp.float32),
    )(value_pad, W_val, b_val.reshape(1, -1))


def _combine_body(attn_ref, qm_ref, id_ref, w_ref, b_ref, o_ref):
    qm = qm_ref[...]
    s = jnp.sum(attn_ref[...] * qm, axis=0)
    cnt = jnp.clip(jnp.sum(qm, axis=0), 1.0, None)
    slots = s / cnt
    o_ref[...] = (jnp.dot(slots, w_ref[...],
                          preferred_element_type=jnp.float32)
                  + b_ref[...] + id_ref[...])


def _combine(attn_pad, qm_pad, ident_pad, W_out, b_out):
    grid = (NQ_PAD // QBLK,)
    return pl.pallas_call(
        _combine_body,
        grid=grid,
        in_specs=[
            pl.BlockSpec((NUM_CAMS, QBLK, EMBED), lambda i: (0, i, 0)),
            pl.BlockSpec((NUM_CAMS, QBLK, EMBED), lambda i: (0, i, 0)),
            pl.BlockSpec((QBLK, EMBED), lambda i: (i, 0)),
            pl.BlockSpec((EMBED, EMBED), lambda i: (0, 0)),
            pl.BlockSpec((1, EMBED), lambda i: (0, 0)),
        ],
        out_specs=pl.BlockSpec((QBLK, EMBED), lambda i: (i, 0)),
        out_shape=jax.ShapeDtypeStruct((NQ_PAD, EMBED), jnp.float32),
    )(attn_pad, qm_pad, ident_pad, W_out, b_out.reshape(1, -1))


def _bilinear(img, x, y):
    N, c, h, w = img.shape
    x0 = jnp.floor(x)
    y0 = jnp.floor(y)
    x1 = x0 + 1.0
    y1 = y0 + 1.0
    wx1 = x - x0
    wx0 = 1.0 - wx1
    wy1 = y - y0
    wy0 = 1.0 - wy1
    flat = img.reshape(N, c, h * w)

    def gather(xi, yi):
        xi_c = jnp.clip(xi, 0, w - 1).astype(jnp.int32)
        yi_c = jnp.clip(yi, 0, h - 1).astype(jnp.int32)
        valid = ((xi >= 0) & (xi <= w - 1) & (yi >= 0) & (yi <= h - 1)).astype(img.dtype)
        idx = yi_c * w + xi_c
        vals = jnp.take_along_axis(
            flat, jnp.broadcast_to(idx[:, None, :], (N, c, idx.shape[1])), axis=2)
        return vals * valid[:, None, :]

    v00 = gather(x0, y0)
    v01 = gather(x1, y0)
    v10 = gather(x0, y1)
    v11 = gather(x1, y1)
    return (v00 * (wx0 * wy0)[:, None, :] + v01 * (wx1 * wy0)[:, None, :]
            + v10 * (wx0 * wy1)[:, None, :] + v11 * (wx1 * wy1)[:, None, :])


def _sample(vproj, loc, aw):
    v = vproj.reshape(NUM_CAMS, NUM_VALUE, HEADS, HEAD_DIM)
    out = jnp.zeros((NUM_CAMS, NQ, HEADS, HEAD_DIM), jnp.float32)
    start = 0
    for l, (h_, w_) in enumerate(SPATIAL_SHAPES):
        img = v[:, start:start + h_ * w_].transpose(0, 2, 3, 1).reshape(
            NUM_CAMS * HEADS, HEAD_DIM, h_, w_)
        start += h_ * w_
        loc_l = loc[:, :, :, l]
        px = (loc_l[..., 0] * w_ - 0.5).transpose(0, 2, 1, 3).reshape(
            NUM_CAMS * HEADS, NQ * POINTS)
        py = (loc_l[..., 1] * h_ - 0.5).transpose(0, 2, 1, 3).reshape(
            NUM_CAMS * HEADS, NQ * POINTS)
        samp = _bilinear(img, px, py).reshape(NUM_CAMS, HEADS, HEAD_DIM, NQ, POINTS)
        wl = jnp.broadcast_to(aw[None, :, :, l].transpose(0, 2, 1, 3),
                              (NUM_CAMS, HEADS, NQ, POINTS))
        out = out + jnp.einsum('bhdqp,bhqp->bqhd', samp, wl)
    return out.reshape(NUM_CAMS, NQ, EMBED)


def kernel(query, value, query_pos, bev_reference_points, lidar2img, img_shape,
           mlvl_feats_spatial_shapes, mlvl_feats_level_start_index,
           W_off, b_off, W_attn, b_attn, W_val, b_val, W_out, b_out):
    B = query.shape[1]
    identity = query
    q = (query + query_pos).reshape(NQ, EMBED)
    q_pad = jnp.pad(q, ((0, NQ_PAD - NQ), (0, 0)))

    off_pad, aw_pad = _frontend(q_pad, W_off, b_off, W_attn, b_attn)
    off = off_pad[:NQ].reshape(NQ, HEADS, LEVELS, POINTS, 2)
    aw = aw_pad[:NQ].reshape(NQ, HEADS, LEVELS, POINTS)

    value_t = value.transpose(1, 0, 2)
    value_pad = jnp.pad(value_t, ((0, 0), (0, NV_PAD - NUM_VALUE), (0, 0)))
    vproj_pad = _valproj(value_pad, W_val, b_val)
    vproj = vproj_pad[:, :NUM_VALUE]

    l2i = jnp.zeros((1, NUM_CAMS, 4, 4), jnp.float32)
    l2i = l2i.at[..., 3, 3].set(1.0).at[..., :3, :4].set(lidar2img)
    ref = bev_reference_points.reshape(NQ, NUM_REFS, 3)
    x = ref[..., 0] * (PC_RANGE[3] - PC_RANGE[0]) + PC_RANGE[0]
    y = ref[..., 1] * (PC_RANGE[4] - PC_RANGE[1]) + PC_RANGE[1]
    z = ref[..., 2] * (PC_RANGE[5] - PC_RANGE[2]) + PC_RANGE[2]
    rp = jnp.stack([x, y, z, jnp.ones_like(x)], -1).reshape(NQ * NUM_REFS, 4)
    rpc = jnp.einsum('nij,qj->nqi', l2i[0], rp)
    eps = 1e-5
    depth_ok = rpc[..., 2] > eps
    uv = rpc[..., 0:2] / jnp.maximum(rpc[..., 2:3], eps)
    u = uv[..., 0] / img_shape[0, :, 1:2]
    vv = uv[..., 1] / img_shape[0, :, 0:1]
    mask = depth_ok & (u > 0.0) & (u < 1.0) & (vv > 0.0) & (vv < 1.0)
    mask = mask.reshape(NUM_CAMS, NQ, NUM_REFS)
    ref_cam = jnp.stack([u, vv], -1).reshape(NUM_CAMS, NQ, NUM_REFS, 2)

    norm = jnp.array([[w_, h_] for (h_, w_) in SPATIAL_SHAPES], jnp.float32)
    ppr = POINTS // NUM_REFS
    offg = off.reshape(NQ, HEADS, LEVELS, NUM_REFS, ppr, 2)
    loc = (ref_cam[:, :, None, None, :, None, :]
           + (offg / norm[None, None, :, None, None, :])[None])
    loc = loc.reshape(NUM_CAMS, NQ, HEADS, LEVELS, POINTS, 2)

    attn = _sample(vproj, loc, aw)

    qmask = (mask.sum(-1) > 0).astype(jnp.float32)
    attn_pad = jnp.pad(attn, ((0, 0), (0, NQ_PAD - NQ), (0, 0)))
    qm_pad = jnp.broadcast_to(
        jnp.pad(qmask, ((0, 0), (0, NQ_PAD - NQ)))[..., None],
        (NUM_CAMS, NQ_PAD, EMBED))
    ident_pad = jnp.pad(identity.reshape(NQ, EMBED), ((0, NQ_PAD - NQ), (0, 0)))

    out_pad = _combine(attn_pad, qm_pad, ident_pad, W_out, b_out)
    return out_pad[:NQ].reshape(NQ, B, EMBED)


# speedup vs baseline: 1326.0542x; 1326.0542x over previous
import functools

import jax
import jax.numpy as jnp
import numpy as np
from jax import lax
from jax.experimental import pallas as pl
from jax.experimental.pallas import tpu as pltpu
from jax.experimental.pallas import tpu_sc as plsc

PC_RANGE = [-51.2, -51.2, -5.0, 51.2, 51.2, 3.0]
SPATIAL_SHAPES = [(58, 100), (29, 50), (15, 25), (8, 13)]
EMBED = 256
HEADS = 8
LEVELS = 4
POINTS = 8
NUM_REFS = 4
NUM_CAMS = 6
NQ = 2500
HEAD_DIM = EMBED // HEADS
NUM_VALUE = sum(h * w for h, w in SPATIAL_SHAPES)

NQ_PAD = 2560
QBLK = 128
NV_PAD = 7936
VBLK = 128


def _frontend_body(q_ref, woff_ref, boff_ref, wattn_ref, battn_ref,
                   off_ref, aw_ref):
    qb = q_ref[...]
    off = jnp.dot(qb, woff_ref[...], preferred_element_type=jnp.float32)
    off_ref[...] = off + boff_ref[...]
    lg = jnp.dot(qb, wattn_ref[...], preferred_element_type=jnp.float32)
    lg = lg + battn_ref[...]
    lp = LEVELS * POINTS
    for h in range(HEADS):
        sl = lg[:, h * lp:(h + 1) * lp]
        m = jnp.max(sl, axis=-1, keepdims=True)
        e = jnp.exp(sl - m)
        s = jnp.sum(e, axis=-1, keepdims=True)
        aw_ref[:, h * lp:(h + 1) * lp] = e / s


def _frontend(q_pad, W_off, b_off, W_attn, b_attn):
    grid = (NQ_PAD // QBLK,)
    return pl.pallas_call(
        _frontend_body,
        grid=grid,
        in_specs=[
            pl.BlockSpec((QBLK, EMBED), lambda i: (i, 0)),
            pl.BlockSpec((EMBED, HEADS * LEVELS * POINTS * 2), lambda i: (0, 0)),
            pl.BlockSpec((1, HEADS * LEVELS * POINTS * 2), lambda i: (0, 0)),
            pl.BlockSpec((EMBED, HEADS * LEVELS * POINTS), lambda i: (0, 0)),
            pl.BlockSpec((1, HEADS * LEVELS * POINTS), lambda i: (0, 0)),
        ],
        out_specs=[
            pl.BlockSpec((QBLK, HEADS * LEVELS * POINTS * 2), lambda i: (i, 0)),
            pl.BlockSpec((QBLK, HEADS * LEVELS * POINTS), lambda i: (i, 0)),
        ],
        out_shape=[
            jax.ShapeDtypeStruct((NQ_PAD, HEADS * LEVELS * POINTS * 2), jnp.float32),
            jax.ShapeDtypeStruct((NQ_PAD, HEADS * LEVELS * POINTS), jnp.float32),
        ],
    )(q_pad, W_off, b_off.reshape(1, -1), W_attn, b_attn.reshape(1, -1))


def _valproj_body(v_ref, w_ref, b_ref, o_ref):
    o_ref[0] = jnp.dot(v_ref[0], w_ref[...],
                       preferred_element_type=jnp.float32) + b_ref[...]


def _valproj(value_pad, W_val, b_val):
    grid = (NUM_CAMS, NV_PAD // VBLK)
    return pl.pallas_call(
        _valproj_body,
        grid=grid,
        in_specs=[
            pl.BlockSpec((1, VBLK, EMBED), lambda c, i: (c, i, 0)),
            pl.BlockSpec((EMBED, EMBED), lambda c, i: (0, 0)),
            pl.BlockSpec((1, EMBED), lambda c, i: (0, 0)),
        ],
        out_specs=pl.BlockSpec((1, VBLK, EMBED), lambda c, i: (c, i, 0)),
        out_shape=jax.ShapeDtypeStruct((NUM_CAMS, NV_PAD, EMBED), jnp.float32),
    )(value_pad, W_val, b_val.reshape(1, -1))


def _combine_body(attn_ref, qm_ref, id_ref, w_ref, b_ref, o_ref):
    qm = qm_ref[...]
    s = jnp.sum(attn_ref[...] * qm, axis=0)
    cnt = jnp.clip(jnp.sum(qm, axis=0), 1.0, None)
    slots = s / cnt
    o_ref[...] = (jnp.dot(slots, w_ref[...],
                          preferred_element_type=jnp.float32)
                  + b_ref[...] + id_ref[...])


def _combine(attn_pad, qm_pad, ident_pad, W_out, b_out):
    grid = (NQ_PAD // QBLK,)
    return pl.pallas_call(
        _combine_body,
        grid=grid,
        in_specs=[
            pl.BlockSpec((NUM_CAMS, QBLK, EMBED), lambda i: (0, i, 0)),
            pl.BlockSpec((NUM_CAMS, QBLK, EMBED), lambda i: (0, i, 0)),
            pl.BlockSpec((QBLK, EMBED), lambda i: (i, 0)),
            pl.BlockSpec((EMBED, EMBED), lambda i: (0, 0)),
            pl.BlockSpec((1, EMBED), lambda i: (0, 0)),
        ],
        out_specs=pl.BlockSpec((QBLK, EMBED), lambda i: (i, 0)),
        out_shape=jax.ShapeDtypeStruct((NQ_PAD, EMBED), jnp.float32),
    )(attn_pad, qm_pad, ident_pad, W_out, b_out.reshape(1, -1))


NITEMS = NUM_CAMS * NQ * HEADS
SAMP = LEVELS * POINTS * 4
SC_NW = 32
SC_CHUNK = 8
ITEMS_PER_W = 3840
NITEMS_PAD = SC_NW * ITEMS_PER_W
NCHUNKS_W = ITEMS_PER_W // SC_CHUNK
FLUSH = 8
TROWS = NUM_CAMS * HEADS * NUM_VALUE


def _build_idx_wgt(loc, aw):
    cam = jnp.arange(NUM_CAMS, dtype=jnp.int32).reshape(NUM_CAMS, 1, 1, 1)
    head = jnp.arange(HEADS, dtype=jnp.int32).reshape(1, 1, HEADS, 1)
    base_ch = (cam * HEADS + head) * NUM_VALUE
    idx_l, wgt_l = [], []
    start = 0
    for l, (h_, w_) in enumerate(SPATIAL_SHAPES):
        px = loc[:, :, :, l, :, 0] * w_ - 0.5
        py = loc[:, :, :, l, :, 1] * h_ - 0.5
        x0 = jnp.floor(px)
        y0 = jnp.floor(py)
        wx1 = px - x0
        wx0 = 1.0 - wx1
        wy1 = py - y0
        wy0 = 1.0 - wy1
        awl = aw[None, :, :, l, :]
        ci, cw = [], []
        for xc, yc, wc in ((x0, y0, wx0 * wy0), (x0 + 1.0, y0, wx1 * wy0),
                           (x0, y0 + 1.0, wx0 * wy1), (x0 + 1.0, y0 + 1.0, wx1 * wy1)):
            valid = ((xc >= 0) & (xc <= w_ - 1) & (yc >= 0) & (yc <= h_ - 1))
            xi = jnp.clip(xc, 0, w_ - 1).astype(jnp.int32)
            yi = jnp.clip(yc, 0, h_ - 1).astype(jnp.int32)
            ci.append(base_ch + start + yi * w_ + xi)
            cw.append(awl * wc * valid.astype(jnp.float32))
        idx_l.append(jnp.stack(ci, -1))
        wgt_l.append(jnp.stack(cw, -1))
        start += h_ * w_
    idx = jnp.stack(idx_l, 3).reshape(NITEMS, SAMP)
    wgt = jnp.stack(wgt_l, 3).reshape(NITEMS, SAMP)
    idx = jnp.pad(idx, ((0, NITEMS_PAD - NITEMS), (0, 0)))
    wgt = jnp.pad(wgt, ((0, NITEMS_PAD - NITEMS), (0, 0)))
    return idx, wgt


_SPLAT_DNUMS = lax.GatherDimensionNumbers(
    offset_dims=(), collapsed_slice_dims=(0,), start_index_map=(0,))


def _sc_sample_body(idx_hbm, wgt_hbm, table_hbm, out_hbm,
                    idx_v, wgt_v, rows_v, out_v, gsem):
    wid = lax.axis_index("s") * 2 + lax.axis_index("c")
    base_item = wid * ITEMS_PER_W

    def chunk_body(g, carry):
        cbase = base_item + g * SC_CHUNK
        pltpu.sync_copy(idx_hbm.at[pl.ds(cbase, SC_CHUNK)], idx_v)
        pltpu.sync_copy(wgt_hbm.at[pl.ds(cbase, SC_CHUNK)], wgt_v)
        for c in range(SC_CHUNK):
            pltpu.async_copy(table_hbm.at[idx_v.at[c]],
                             rows_v.at[pl.ds(c * SAMP, SAMP)], gsem)
        for c in range(SC_CHUNK):
            pltpu.make_async_copy(table_hbm.at[idx_v.at[c]],
                                  rows_v.at[pl.ds(c * SAMP, SAMP)], gsem).wait()

        def item_body(i, carry2):
            acc0 = jnp.zeros((16,), jnp.float32)
            acc1 = jnp.zeros((16,), jnp.float32)
            rbase = i * SAMP
            for g16 in range(SAMP // 16):
                wv = wgt_v[i, pl.ds(g16 * 16, 16)]
                for j in range(16):
                    r = rbase + g16 * 16 + j
                    wb = lax.gather(
                        wv, jnp.full((16, 1), j, jnp.int32), _SPLAT_DNUMS, (1,),
                        mode=lax.GatherScatterMode.PROMISE_IN_BOUNDS)
                    acc0 = acc0 + wb * rows_v[r, pl.ds(0, 16)]
                    acc1 = acc1 + wb * rows_v[r, pl.ds(16, 16)]
            orow = (g % FLUSH) * SC_CHUNK + i
            out_v[orow, pl.ds(0, 16)] = acc0
            out_v[orow, pl.ds(16, 16)] = acc1
            return carry2

        lax.fori_loop(0, SC_CHUNK, item_body, 0)

        @pl.when(g % FLUSH == FLUSH - 1)
        def _():
            obase = base_item + (g - (FLUSH - 1)) * SC_CHUNK
            pltpu.sync_copy(out_v, out_hbm.at[pl.ds(obase, SC_CHUNK * FLUSH)])

        return carry

    lax.fori_loop(0, NCHUNKS_W, chunk_body, 0)


def _sc_sample(idx, wgt, table):
    fn = functools.partial(
        pl.kernel,
        mesh=plsc.VectorSubcoreMesh(core_axis_name="c", subcore_axis_name="s"),
        out_type=jax.ShapeDtypeStruct((NITEMS_PAD, HEAD_DIM), jnp.float32),
        scratch_types=[
            pltpu.VMEM((SC_CHUNK, SAMP), jnp.int32),
            pltpu.VMEM((SC_CHUNK, SAMP), jnp.float32),
            pltpu.VMEM((SC_CHUNK * SAMP, HEAD_DIM), jnp.float32),
            pltpu.VMEM((FLUSH * SC_CHUNK, HEAD_DIM), jnp.float32),
            pltpu.SemaphoreType.DMA,
        ],
        compiler_params=pltpu.CompilerParams(use_tc_tiling_on_sc=False),
    )(_sc_sample_body)
    return fn(idx, wgt, table)


def _bilinear(img, x, y):
    N, c, h, w = img.shape
    x0 = jnp.floor(x)
    y0 = jnp.floor(y)
    x1 = x0 + 1.0
    y1 = y0 + 1.0
    wx1 = x - x0
    wx0 = 1.0 - wx1
    wy1 = y - y0
    wy0 = 1.0 - wy1
    flat = img.reshape(N, c, h * w)

    def gather(xi, yi):
        xi_c = jnp.clip(xi, 0, w - 1).astype(jnp.int32)
        yi_c = jnp.clip(yi, 0, h - 1).astype(jnp.int32)
        valid = ((xi >= 0) & (xi <= w - 1) & (yi >= 0) & (yi <= h - 1)).astype(img.dtype)
        idx = yi_c * w + xi_c
        vals = jnp.take_along_axis(
            flat, jnp.broadcast_to(idx[:, None, :], (N, c, idx.shape[1])), axis=2)
        return vals * valid[:, None, :]

    v00 = gather(x0, y0)
    v01 = gather(x1, y0)
    v10 = gather(x0, y1)
    v11 = gather(x1, y1)
    return (v00 * (wx0 * wy0)[:, None, :] + v01 * (wx1 * wy0)[:, None, :]
            + v10 * (wx0 * wy1)[:, None, :] + v11 * (wx1 * wy1)[:, None, :])


def _sample(vproj, loc, aw):
    v = vproj.reshape(NUM_CAMS, NUM_VALUE, HEADS, HEAD_DIM)
    out = jnp.zeros((NUM_CAMS, NQ, HEADS, HEAD_DIM), jnp.float32)
    start = 0
    for l, (h_, w_) in enumerate(SPATIAL_SHAPES):
        img = v[:, start:start + h_ * w_].transpose(0, 2, 3, 1).reshape(
            NUM_CAMS * HEADS, HEAD_DIM, h_, w_)
        start += h_ * w_
        loc_l = loc[:, :, :, l]
        px = (loc_l[..., 0] * w_ - 0.5).transpose(0, 2, 1, 3).reshape(
            NUM_CAMS * HEADS, NQ * POINTS)
        py = (loc_l[..., 1] * h_ - 0.5).transpose(0, 2, 1, 3).reshape(
            NUM_CAMS * HEADS, NQ * POINTS)
        samp = _bilinear(img, px, py).reshape(NUM_CAMS, HEADS, HEAD_DIM, NQ, POINTS)
        wl = jnp.broadcast_to(aw[None, :, :, l].transpose(0, 2, 1, 3),
                              (NUM_CAMS, HEADS, NQ, POINTS))
        out = out + jnp.einsum('bhdqp,bhqp->bqhd', samp, wl)
    return out.reshape(NUM_CAMS, NQ, EMBED)


def kernel(query, value, query_pos, bev_reference_points, lidar2img, img_shape,
           mlvl_feats_spatial_shapes, mlvl_feats_level_start_index,
           W_off, b_off, W_attn, b_attn, W_val, b_val, W_out, b_out):
    B = query.shape[1]
    identity = query
    q = (query + query_pos).reshape(NQ, EMBED)
    q_pad = jnp.pad(q, ((0, NQ_PAD - NQ), (0, 0)))

    off_pad, aw_pad = _frontend(q_pad, W_off, b_off, W_attn, b_attn)
    off = off_pad[:NQ].reshape(NQ, HEADS, LEVELS, POINTS, 2)
    aw = aw_pad[:NQ].reshape(NQ, HEADS, LEVELS, POINTS)

    value_t = value.transpose(1, 0, 2)
    value_pad = jnp.pad(value_t, ((0, 0), (0, NV_PAD - NUM_VALUE), (0, 0)))
    vproj_pad = _valproj(value_pad, W_val, b_val)
    vproj = vproj_pad[:, :NUM_VALUE]

    l2i = jnp.zeros((1, NUM_CAMS, 4, 4), jnp.float32)
    l2i = l2i.at[..., 3, 3].set(1.0).at[..., :3, :4].set(lidar2img)
    ref = bev_reference_points.reshape(NQ, NUM_REFS, 3)
    x = ref[..., 0] * (PC_RANGE[3] - PC_RANGE[0]) + PC_RANGE[0]
    y = ref[..., 1] * (PC_RANGE[4] - PC_RANGE[1]) + PC_RANGE[1]
    z = ref[..., 2] * (PC_RANGE[5] - PC_RANGE[2]) + PC_RANGE[2]
    rp = jnp.stack([x, y, z, jnp.ones_like(x)], -1).reshape(NQ * NUM_REFS, 4)
    rpc = jnp.einsum('nij,qj->nqi', l2i[0], rp)
    eps = 1e-5
    depth_ok = rpc[..., 2] > eps
    uv = rpc[..., 0:2] / jnp.maximum(rpc[..., 2:3], eps)
    u = uv[..., 0] / img_shape[0, :, 1:2]
    vv = uv[..., 1] / img_shape[0, :, 0:1]
    mask = depth_ok & (u > 0.0) & (u < 1.0) & (vv > 0.0) & (vv < 1.0)
    mask = mask.reshape(NUM_CAMS, NQ, NUM_REFS)
    ref_cam = jnp.stack([u, vv], -1).reshape(NUM_CAMS, NQ, NUM_REFS, 2)

    norm = jnp.array([[w_, h_] for (h_, w_) in SPATIAL_SHAPES], jnp.float32)
    ppr = POINTS // NUM_REFS
    offg = off.reshape(NQ, HEADS, LEVELS, NUM_REFS, ppr, 2)
    loc = (ref_cam[:, :, None, None, :, None, :]
           + (offg / norm[None, None, :, None, None, :])[None])
    loc = loc.reshape(NUM_CAMS, NQ, HEADS, LEVELS, POINTS, 2)

    table = vproj.reshape(NUM_CAMS, NUM_VALUE, HEADS, HEAD_DIM).transpose(
        0, 2, 1, 3).reshape(TROWS, HEAD_DIM)
    idx, wgt = _build_idx_wgt(loc, aw)
    attn_flat = _sc_sample(idx, wgt, table)
    attn = attn_flat[:NITEMS].reshape(NUM_CAMS, NQ, EMBED)

    qmask = (mask.sum(-1) > 0).astype(jnp.float32)
    attn_pad = jnp.pad(attn, ((0, 0), (0, NQ_PAD - NQ), (0, 0)))
    qm_pad = jnp.broadcast_to(
        jnp.pad(qmask, ((0, 0), (0, NQ_PAD - NQ)))[..., None],
        (NUM_CAMS, NQ_PAD, EMBED))
    ident_pad = jnp.pad(identity.reshape(NQ, EMBED), ((0, NQ_PAD - NQ), (0, 0)))

    out_pad = _combine(attn_pad, qm_pad, ident_pad, W_out, b_out)
    return out_pad[:NQ].reshape(NQ, B, EMBED)


# speedup vs baseline: 1443.5189x; 1.0886x over previous
import functools

import jax
import jax.numpy as jnp
import numpy as np
from jax import lax
from jax.experimental import pallas as pl
from jax.experimental.pallas import tpu as pltpu
from jax.experimental.pallas import tpu_sc as plsc

PC_RANGE = [-51.2, -51.2, -5.0, 51.2, 51.2, 3.0]
SPATIAL_SHAPES = [(58, 100), (29, 50), (15, 25), (8, 13)]
EMBED = 256
HEADS = 8
LEVELS = 4
POINTS = 8
NUM_REFS = 4
NUM_CAMS = 6
NQ = 2500
HEAD_DIM = EMBED // HEADS
NUM_VALUE = sum(h * w for h, w in SPATIAL_SHAPES)

NQ_PAD = 2560
QBLK = 128
NV_PAD = 7936
VBLK = 128


def _frontend_body(q_ref, woff_ref, boff_ref, wattn_ref, battn_ref,
                   off_ref, aw_ref):
    qb = q_ref[...]
    off = jnp.dot(qb, woff_ref[...], preferred_element_type=jnp.float32)
    off_ref[...] = off + boff_ref[...]
    lg = jnp.dot(qb, wattn_ref[...], preferred_element_type=jnp.float32)
    lg = lg + battn_ref[...]
    lp = LEVELS * POINTS
    for h in range(HEADS):
        sl = lg[:, h * lp:(h + 1) * lp]
        m = jnp.max(sl, axis=-1, keepdims=True)
        e = jnp.exp(sl - m)
        s = jnp.sum(e, axis=-1, keepdims=True)
        aw_ref[:, h * lp:(h + 1) * lp] = e / s


def _frontend(q_pad, W_off, b_off, W_attn, b_attn):
    grid = (NQ_PAD // QBLK,)
    return pl.pallas_call(
        _frontend_body,
        grid=grid,
        in_specs=[
            pl.BlockSpec((QBLK, EMBED), lambda i: (i, 0)),
            pl.BlockSpec((EMBED, HEADS * LEVELS * POINTS * 2), lambda i: (0, 0)),
            pl.BlockSpec((1, HEADS * LEVELS * POINTS * 2), lambda i: (0, 0)),
            pl.BlockSpec((EMBED, HEADS * LEVELS * POINTS), lambda i: (0, 0)),
            pl.BlockSpec((1, HEADS * LEVELS * POINTS), lambda i: (0, 0)),
        ],
        out_specs=[
            pl.BlockSpec((QBLK, HEADS * LEVELS * POINTS * 2), lambda i: (i, 0)),
            pl.BlockSpec((QBLK, HEADS * LEVELS * POINTS), lambda i: (i, 0)),
        ],
        out_shape=[
            jax.ShapeDtypeStruct((NQ_PAD, HEADS * LEVELS * POINTS * 2), jnp.float32),
            jax.ShapeDtypeStruct((NQ_PAD, HEADS * LEVELS * POINTS), jnp.float32),
        ],
    )(q_pad, W_off, b_off.reshape(1, -1), W_attn, b_attn.reshape(1, -1))


def _valproj_body(v_ref, w_ref, b_ref, o_ref):
    o_ref[0] = jnp.dot(v_ref[0], w_ref[...],
                       preferred_element_type=jnp.float32) + b_ref[...]


def _valproj(value_pad, W_val, b_val):
    grid = (NUM_CAMS, NV_PAD // VBLK)
    return pl.pallas_call(
        _valproj_body,
        grid=grid,
        in_specs=[
            pl.BlockSpec((1, VBLK, EMBED), lambda c, i: (c, i, 0)),
            pl.BlockSpec((EMBED, EMBED), lambda c, i: (0, 0)),
            pl.BlockSpec((1, EMBED), lambda c, i: (0, 0)),
        ],
        out_specs=pl.BlockSpec((1, VBLK, EMBED), lambda c, i: (c, i, 0)),
        out_shape=jax.ShapeDtypeStruct((NUM_CAMS, NV_PAD, EMBED), jnp.float32),
    )(value_pad, W_val, b_val.reshape(1, -1))


def _combine_body(attn_ref, qm_ref, id_ref, w_ref, b_ref, o_ref):
    qm = qm_ref[...]
    s = jnp.sum(attn_ref[...] * qm, axis=0)
    cnt = jnp.clip(jnp.sum(qm, axis=0), 1.0, None)
    slots = s / cnt
    o_ref[...] = (jnp.dot(slots, w_ref[...],
                          preferred_element_type=jnp.float32)
                  + b_ref[...] + id_ref[...])


def _combine(attn_pad, qm_pad, ident_pad, W_out, b_out):
    grid = (NQ_PAD // QBLK,)
    return pl.pallas_call(
        _combine_body,
        grid=grid,
        in_specs=[
            pl.BlockSpec((NUM_CAMS, QBLK, EMBED), lambda i: (0, i, 0)),
            pl.BlockSpec((NUM_CAMS, QBLK, EMBED), lambda i: (0, i, 0)),
            pl.BlockSpec((QBLK, EMBED), lambda i: (i, 0)),
            pl.BlockSpec((EMBED, EMBED), lambda i: (0, 0)),
            pl.BlockSpec((1, EMBED), lambda i: (0, 0)),
        ],
        out_specs=pl.BlockSpec((QBLK, EMBED), lambda i: (i, 0)),
        out_shape=jax.ShapeDtypeStruct((NQ_PAD, EMBED), jnp.float32),
    )(attn_pad, qm_pad, ident_pad, W_out, b_out.reshape(1, -1))


NITEMS = NUM_CAMS * NQ * HEADS
SAMP = LEVELS * POINTS * 4
SC_NW = 32
SC_CHUNK = 8
ITEMS_PER_W = 3840
NITEMS_PAD = SC_NW * ITEMS_PER_W
NCHUNKS_W = ITEMS_PER_W // SC_CHUNK
FLUSH = 8
TROWS = NUM_CAMS * HEADS * NUM_VALUE


def _build_idx_wgt(loc, aw):
    cam = jnp.arange(NUM_CAMS, dtype=jnp.int32).reshape(NUM_CAMS, 1, 1, 1)
    head = jnp.arange(HEADS, dtype=jnp.int32).reshape(1, 1, HEADS, 1)
    base_ch = (cam * HEADS + head) * NUM_VALUE
    idx_l, wgt_l = [], []
    start = 0
    for l, (h_, w_) in enumerate(SPATIAL_SHAPES):
        px = loc[:, :, :, l, :, 0] * w_ - 0.5
        py = loc[:, :, :, l, :, 1] * h_ - 0.5
        x0 = jnp.floor(px)
        y0 = jnp.floor(py)
        wx1 = px - x0
        wx0 = 1.0 - wx1
        wy1 = py - y0
        wy0 = 1.0 - wy1
        awl = aw[None, :, :, l, :]
        ci, cw = [], []
        for xc, yc, wc in ((x0, y0, wx0 * wy0), (x0 + 1.0, y0, wx1 * wy0),
                           (x0, y0 + 1.0, wx0 * wy1), (x0 + 1.0, y0 + 1.0, wx1 * wy1)):
            valid = ((xc >= 0) & (xc <= w_ - 1) & (yc >= 0) & (yc <= h_ - 1))
            xi = jnp.clip(xc, 0, w_ - 1).astype(jnp.int32)
            yi = jnp.clip(yc, 0, h_ - 1).astype(jnp.int32)
            ci.append(base_ch + start + yi * w_ + xi)
            cw.append(awl * wc * valid.astype(jnp.float32))
        idx_l.append(jnp.stack(ci, -1))
        wgt_l.append(jnp.stack(cw, -1))
        start += h_ * w_
    idx = jnp.stack(idx_l, 3).reshape(NITEMS, SAMP)
    wgt = jnp.stack(wgt_l, 3).reshape(NITEMS, SAMP)
    idx = jnp.pad(idx, ((0, NITEMS_PAD - NITEMS), (0, 0)))
    wgt = jnp.pad(wgt, ((0, NITEMS_PAD - NITEMS), (0, 0)))
    return idx, wgt


_SPLAT_DNUMS = lax.GatherDimensionNumbers(
    offset_dims=(), collapsed_slice_dims=(0,), start_index_map=(0,))


def _sc_sample_body(idx_hbm, wgt_hbm, table_hbm, out_hbm,
                    idx_v, wgt_v, rows_v, out_v, dsem, gsem):
    wid = lax.axis_index("s") * 2 + lax.axis_index("c")
    base_item = wid * ITEMS_PER_W

    def idx_copies(g, b):
        cbase = base_item + g * SC_CHUNK
        return (
            pltpu.make_async_copy(idx_hbm.at[pl.ds(cbase, SC_CHUNK)],
                                  idx_v.at[b], dsem.at[b]),
            pltpu.make_async_copy(wgt_hbm.at[pl.ds(cbase, SC_CHUNK)],
                                  wgt_v.at[b], dsem.at[b]),
        )

    def gather_copies(b):
        return [pltpu.make_async_copy(table_hbm.at[idx_v.at[b, c]],
                                      rows_v.at[b, pl.ds(c * SAMP, SAMP)],
                                      gsem.at[b])
                for c in range(SC_CHUNK)]

    for cp in idx_copies(0, 0):
        cp.start()
    for cp in idx_copies(0, 0):
        cp.wait()
    for cp in gather_copies(0):
        cp.start()
    for cp in idx_copies(1, 1):
        cp.start()

    def chunk_body(g, carry):
        b = lax.rem(g, 2)
        bn = 1 - b
        for cp in gather_copies(b):
            cp.wait()

        @pl.when(g + 1 < NCHUNKS_W)
        def _():
            for cp in idx_copies(g + 1, bn):
                cp.wait()
            for cp in gather_copies(bn):
                cp.start()

        def item_body(i, carry2):
            acc0 = jnp.zeros((16,), jnp.float32)
            acc1 = jnp.zeros((16,), jnp.float32)
            rbase = i * SAMP
            for g16 in range(SAMP // 16):
                wv = wgt_v[b, i, pl.ds(g16 * 16, 16)]
                for j in range(16):
                    r = rbase + g16 * 16 + j
                    wb = lax.gather(
                        wv, jnp.full((16, 1), j, jnp.int32), _SPLAT_DNUMS, (1,),
                        mode=lax.GatherScatterMode.PROMISE_IN_BOUNDS)
                    acc0 = acc0 + wb * rows_v[b, r, pl.ds(0, 16)]
                    acc1 = acc1 + wb * rows_v[b, r, pl.ds(16, 16)]
            orow = (g % FLUSH) * SC_CHUNK + i
            out_v[orow, pl.ds(0, 16)] = acc0
            out_v[orow, pl.ds(16, 16)] = acc1
            return carry2

        lax.fori_loop(0, SC_CHUNK, item_body, 0)

        @pl.when(g + 2 < NCHUNKS_W)
        def _():
            for cp in idx_copies(g + 2, b):
                cp.start()

        @pl.when(g % FLUSH == FLUSH - 1)
        def _():
            obase = base_item + (g - (FLUSH - 1)) * SC_CHUNK
            pltpu.sync_copy(out_v, out_hbm.at[pl.ds(obase, SC_CHUNK * FLUSH)])

        return carry

    lax.fori_loop(0, NCHUNKS_W, chunk_body, 0)


def _sc_sample(idx, wgt, table):
    fn = functools.partial(
        pl.kernel,
        mesh=plsc.VectorSubcoreMesh(core_axis_name="c", subcore_axis_name="s"),
        out_type=jax.ShapeDtypeStruct((NITEMS_PAD, HEAD_DIM), jnp.float32),
        scratch_types=[
            pltpu.VMEM((2, SC_CHUNK, SAMP), jnp.int32),
            pltpu.VMEM((2, SC_CHUNK, SAMP), jnp.float32),
            pltpu.VMEM((2, SC_CHUNK * SAMP, HEAD_DIM), jnp.float32),
            pltpu.VMEM((FLUSH * SC_CHUNK, HEAD_DIM), jnp.float32),
            pltpu.SemaphoreType.DMA((2,)),
            pltpu.SemaphoreType.DMA((2,)),
        ],
        compiler_params=pltpu.CompilerParams(use_tc_tiling_on_sc=False),
    )(_sc_sample_body)
    return fn(idx, wgt, table)


def _bilinear(img, x, y):
    N, c, h, w = img.shape
    x0 = jnp.floor(x)
    y0 = jnp.floor(y)
    x1 = x0 + 1.0
    y1 = y0 + 1.0
    wx1 = x - x0
    wx0 = 1.0 - wx1
    wy1 = y - y0
    wy0 = 1.0 - wy1
    flat = img.reshape(N, c, h * w)

    def gather(xi, yi):
        xi_c = jnp.clip(xi, 0, w - 1).astype(jnp.int32)
        yi_c = jnp.clip(yi, 0, h - 1).astype(jnp.int32)
        valid = ((xi >= 0) & (xi <= w - 1) & (yi >= 0) & (yi <= h - 1)).astype(img.dtype)
        idx = yi_c * w + xi_c
        vals = jnp.take_along_axis(
            flat, jnp.broadcast_to(idx[:, None, :], (N, c, idx.shape[1])), axis=2)
        return vals * valid[:, None, :]

    v00 = gather(x0, y0)
    v01 = gather(x1, y0)
    v10 = gather(x0, y1)
    v11 = gather(x1, y1)
    return (v00 * (wx0 * wy0)[:, None, :] + v01 * (wx1 * wy0)[:, None, :]
            + v10 * (wx0 * wy1)[:, None, :] + v11 * (wx1 * wy1)[:, None, :])


def _sample(vproj, loc, aw):
    v = vproj.reshape(NUM_CAMS, NUM_VALUE, HEADS, HEAD_DIM)
    out = jnp.zeros((NUM_CAMS, NQ, HEADS, HEAD_DIM), jnp.float32)
    start = 0
    for l, (h_, w_) in enumerate(SPATIAL_SHAPES):
        img = v[:, start:start + h_ * w_].transpose(0, 2, 3, 1).reshape(
            NUM_CAMS * HEADS, HEAD_DIM, h_, w_)
        start += h_ * w_
        loc_l = loc[:, :, :, l]
        px = (loc_l[..., 0] * w_ - 0.5).transpose(0, 2, 1, 3).reshape(
            NUM_CAMS * HEADS, NQ * POINTS)
        py = (loc_l[..., 1] * h_ - 0.5).transpose(0, 2, 1, 3).reshape(
            NUM_CAMS * HEADS, NQ * POINTS)
        samp = _bilinear(img, px, py).reshape(NUM_CAMS, HEADS, HEAD_DIM, NQ, POINTS)
        wl = jnp.broadcast_to(aw[None, :, :, l].transpose(0, 2, 1, 3),
                              (NUM_CAMS, HEADS, NQ, POINTS))
        out = out + jnp.einsum('bhdqp,bhqp->bqhd', samp, wl)
    return out.reshape(NUM_CAMS, NQ, EMBED)


def kernel(query, value, query_pos, bev_reference_points, lidar2img, img_shape,
           mlvl_feats_spatial_shapes, mlvl_feats_level_start_index,
           W_off, b_off, W_attn, b_attn, W_val, b_val, W_out, b_out):
    B = query.shape[1]
    identity = query
    q = (query + query_pos).reshape(NQ, EMBED)
    q_pad = jnp.pad(q, ((0, NQ_PAD - NQ), (0, 0)))

    off_pad, aw_pad = _frontend(q_pad, W_off, b_off, W_attn, b_attn)
    off = off_pad[:NQ].reshape(NQ, HEADS, LEVELS, POINTS, 2)
    aw = aw_pad[:NQ].reshape(NQ, HEADS, LEVELS, POINTS)

    value_t = value.transpose(1, 0, 2)
    value_pad = jnp.pad(value_t, ((0, 0), (0, NV_PAD - NUM_VALUE), (0, 0)))
    vproj_pad = _valproj(value_pad, W_val, b_val)
    vproj = vproj_pad[:, :NUM_VALUE]

    l2i = jnp.zeros((1, NUM_CAMS, 4, 4), jnp.float32)
    l2i = l2i.at[..., 3, 3].set(1.0).at[..., :3, :4].set(lidar2img)
    ref = bev_reference_points.reshape(NQ, NUM_REFS, 3)
    x = ref[..., 0] * (PC_RANGE[3] - PC_RANGE[0]) + PC_RANGE[0]
    y = ref[..., 1] * (PC_RANGE[4] - PC_RANGE[1]) + PC_RANGE[1]
    z = ref[..., 2] * (PC_RANGE[5] - PC_RANGE[2]) + PC_RANGE[2]
    rp = jnp.stack([x, y, z, jnp.ones_like(x)], -1).reshape(NQ * NUM_REFS, 4)
    rpc = jnp.einsum('nij,qj->nqi', l2i[0], rp)
    eps = 1e-5
    depth_ok = rpc[..., 2] > eps
    uv = rpc[..., 0:2] / jnp.maximum(rpc[..., 2:3], eps)
    u = uv[..., 0] / img_shape[0, :, 1:2]
    vv = uv[..., 1] / img_shape[0, :, 0:1]
    mask = depth_ok & (u > 0.0) & (u < 1.0) & (vv > 0.0) & (vv < 1.0)
    mask = mask.reshape(NUM_CAMS, NQ, NUM_REFS)
    ref_cam = jnp.stack([u, vv], -1).reshape(NUM_CAMS, NQ, NUM_REFS, 2)

    norm = jnp.array([[w_, h_] for (h_, w_) in SPATIAL_SHAPES], jnp.float32)
    ppr = POINTS // NUM_REFS
    offg = off.reshape(NQ, HEADS, LEVELS, NUM_REFS, ppr, 2)
    loc = (ref_cam[:, :, None, None, :, None, :]
           + (offg / norm[None, None, :, None, None, :])[None])
    loc = loc.reshape(NUM_CAMS, NQ, HEADS, LEVELS, POINTS, 2)

    table = vproj.reshape(NUM_CAMS, NUM_VALUE, HEADS, HEAD_DIM).transpose(
        0, 2, 1, 3).reshape(TROWS, HEAD_DIM)
    idx, wgt = _build_idx_wgt(loc, aw)
    attn_flat = _sc_sample(idx, wgt, table)
    attn = attn_flat[:NITEMS].reshape(NUM_CAMS, NQ, EMBED)

    qmask = (mask.sum(-1) > 0).astype(jnp.float32)
    attn_pad = jnp.pad(attn, ((0, 0), (0, NQ_PAD - NQ), (0, 0)))
    qm_pad = jnp.broadcast_to(
        jnp.pad(qmask, ((0, 0), (0, NQ_PAD - NQ)))[..., None],
        (NUM_CAMS, NQ_PAD, EMBED))
    ident_pad = jnp.pad(identity.reshape(NQ, EMBED), ((0, NQ_PAD - NQ), (0, 0)))

    out_pad = _combine(attn_pad, qm_pad, ident_pad, W_out, b_out)
    return out_pad[:NQ].reshape(NQ, B, EMBED)
